# Initial kernel scaffold; baseline (speedup 1.0000x reference)
import jax
import jax.numpy as jnp
from jax.experimental import pallas as pl


def kernel(input, W):
    raise NotImplementedError("write your pallas kernel here")


# speedup vs baseline: 2.3514x; 2.3514x over previous
import functools

import jax
import jax.numpy as jnp
from jax import lax
from jax.experimental import pallas as pl
from jax.experimental.pallas import tpu as pltpu
from jax.experimental.pallas import tpu_sc as plsc

TOKENS = 8192
DM = 4096
NE = 64
TOPK = 8
NWORK = 32
CHUNK = TOKENS * TOPK // NWORK
BT = TOKENS // NWORK


def _gate_block(x_ref, w_ref, gate_ref, idx_ref, cumx_ref,
                tpe_ref, laux_ref, me_acc, hist_acc):
    i = pl.program_id(0)

    @pl.when(i == 0)
    def _init():
        me_acc[...] = jnp.zeros_like(me_acc)
        hist_acc[...] = jnp.zeros_like(hist_acc)

    cumx_ref[0, 0, :] = hist_acc[0, :].astype(jnp.int32)

    logits = lax.dot_general(x_ref[...], w_ref[...],
                             (((1,), (1,)), ((), ())),
                             preferred_element_type=jnp.float32)

    iota_e = lax.broadcasted_iota(jnp.int32, (BT, NE), 1)
    cur = logits
    sel_acc = jnp.zeros((BT, NE), jnp.float32)
    row_max = None
    gates = []
    idxs = []
    for j in range(TOPK):
        m = jnp.max(cur, axis=1, keepdims=True)
        if j == 0:
            row_max = m
        eq = cur == m
        idx = jnp.min(jnp.where(eq, iota_e, NE), axis=1, keepdims=True)
        sel = iota_e == idx
        gates.append(m)
        idxs.append(idx)
        cur = jnp.where(sel, -jnp.inf, cur)
        sel_acc = sel_acc + sel.astype(jnp.float32)

    gate_ref[...] = jnp.concatenate(gates, axis=1)
    idx_ref[...] = jnp.concatenate(idxs, axis=1)

    ex = jnp.exp(logits - row_max)
    scores = ex / jnp.sum(ex, axis=1, keepdims=True)
    me_acc[0, :] = me_acc[0, :] + jnp.sum(scores, axis=0)
    hist_acc[0, :] = hist_acc[0, :] + jnp.sum(sel_acc, axis=0)

    @pl.when(i == NWORK - 1)
    def _final():
        tpe_f = hist_acc[0, :]
        tpe_ref[0, :] = tpe_f.astype(jnp.int32)
        me = me_acc[0, :] * (1.0 / TOKENS)
        ce = tpe_f * (1.0 / TOKENS)
        laux_ref[...] = jnp.sum(me * ce).reshape(1, 1) * (NE / TOPK)


_gate_call = pl.pallas_call(
    _gate_block,
    grid=(NWORK,),
    in_specs=[
        pl.BlockSpec((BT, DM), lambda i: (i, 0)),
        pl.BlockSpec((NE, DM), lambda i: (0, 0)),
    ],
    out_specs=[
        pl.BlockSpec((BT, TOPK), lambda i: (i, 0)),
        pl.BlockSpec((BT, TOPK), lambda i: (i, 0)),
        pl.BlockSpec((1, 1, NE), lambda i: (i, 0, 0)),
        pl.BlockSpec((1, NE), lambda i: (0, 0)),
        pl.BlockSpec((1, 1), lambda i: (0, 0)),
    ],
    out_shape=[
        jax.ShapeDtypeStruct((TOKENS, TOPK), jnp.float32),
        jax.ShapeDtypeStruct((TOKENS, TOPK), jnp.int32),
        jax.ShapeDtypeStruct((NWORK, 1, NE), jnp.int32),
        jax.ShapeDtypeStruct((1, NE), jnp.int32),
        jax.ShapeDtypeStruct((1, 1), jnp.float32),
    ],
    scratch_shapes=[
        pltpu.VMEM((1, NE), jnp.float32),
        pltpu.VMEM((1, NE), jnp.float32),
    ],
)


def _rank_body(te_hbm, tpe_hbm, cumx_hbm, pos_hbm, val_hbm, bins_hbm,
               te_v, pos_v, val_v, cnt_v, tmp_v, bin_v):
    c = lax.axis_index("c")
    s = lax.axis_index("s")
    wid = s * 2 + c

    pltpu.sync_copy(te_hbm.at[pl.ds(wid * 16, 16)], te_v)
    pltpu.sync_copy(tpe_hbm.at[0], tmp_v)
    pltpu.sync_copy(cumx_hbm.at[wid, 0], cnt_v)

    carry = jnp.int32(0)
    for t in range(4):
        sl = pl.ds(t * 16, 16)
        v = tmp_v[sl]
        incl = plsc.cumsum(v)
        cnt_v[sl] = cnt_v[sl] + (carry + incl - v)
        bin_v[sl] = carry + incl
        carry = carry + jnp.sum(v)

    @pl.when(wid == 0)
    def _bins_out():
        pltpu.sync_copy(bin_v, bins_hbm)

    base = wid * CHUNK
    for r in range(16):
        for q in range(8):
            sl = pl.ds(q * 16, 16)
            keys = te_v[r, sl]
            cnt, last = plsc.scan_count(keys)
            b = plsc.load_gather(cnt_v, [keys])
            pos_v[r, sl] = b + cnt - 1
            plsc.store_scatter(cnt_v, [keys], b + cnt, mask=last)
            val_v[r, sl] = (base + r * 128 + q * 16
                            + lax.iota(jnp.int32, 16)) * NE + keys

    pltpu.sync_copy(pos_v, pos_hbm.at[pl.ds(wid * 16, 16)])
    pltpu.sync_copy(val_v, val_hbm.at[pl.ds(wid * 16, 16)])


def _permute_body(pos_hbm, val_hbm, idxout_hbm, binout_hbm,
                  pblk, vblk, idx_loc, bin_loc):
    c = lax.axis_index("c")
    s = lax.axis_index("s")
    wid = s * 2 + c
    lo = wid * CHUNK

    def blk(b, _):
        pltpu.sync_copy(pos_hbm.at[pl.ds(b * 32, 32)], pblk)
        pltpu.sync_copy(val_hbm.at[pl.ds(b * 32, 32)], vblk)
        for r in range(32):
            for q in range(8):
                sl = pl.ds(q * 16, 16)
                p = pblk[r, sl] - lo
                m = (p >= 0) & (p < CHUNK)
                ps = jnp.where(m, p, 0)
                v = vblk[r, sl]
                plsc.store_scatter(idx_loc, [ps],
                                   lax.shift_right_logical(v, 6), mask=m)
                plsc.store_scatter(bin_loc, [ps], v & (NE - 1), mask=m)
        return 0

    lax.fori_loop(0, 16, blk, 0)
    pltpu.sync_copy(idx_loc, idxout_hbm.at[pl.ds(lo, CHUNK)])
    pltpu.sync_copy(bin_loc, binout_hbm.at[pl.ds(lo, CHUNK)])


@functools.cache
def _sc_calls():
    mesh = plsc.VectorSubcoreMesh(core_axis_name="c", subcore_axis_name="s",
                                  num_cores=2, num_subcores=16)
    cp = pltpu.CompilerParams(needs_layout_passes=False)
    rank_call = pl.kernel(
        _rank_body,
        out_type=[
            jax.ShapeDtypeStruct((512, 128), jnp.int32),
            jax.ShapeDtypeStruct((512, 128), jnp.int32),
            jax.ShapeDtypeStruct((NE,), jnp.int32),
        ],
        mesh=mesh,
        compiler_params=cp,
        scratch_types=[
            pltpu.VMEM((16, 128), jnp.int32),
            pltpu.VMEM((16, 128), jnp.int32),
            pltpu.VMEM((16, 128), jnp.int32),
            pltpu.VMEM((NE,), jnp.int32),
            pltpu.VMEM((NE,), jnp.int32),
            pltpu.VMEM((NE,), jnp.int32),
        ],
    )
    permute_call = pl.kernel(
        _permute_body,
        out_type=[
            jax.ShapeDtypeStruct((TOKENS * TOPK,), jnp.int32),
            jax.ShapeDtypeStruct((TOKENS * TOPK,), jnp.int32),
        ],
        mesh=mesh,
        compiler_params=cp,
        scratch_types=[
            pltpu.VMEM((32, 128), jnp.int32),
            pltpu.VMEM((32, 128), jnp.int32),
            pltpu.VMEM((CHUNK,), jnp.int32),
            pltpu.VMEM((CHUNK,), jnp.int32),
        ],
    )
    return rank_call, permute_call


def kernel(input, W):
    gate, idx, cumx, tpe, laux = _gate_call(input, W)
    rank_call, permute_call = _sc_calls()
    te2d = idx.reshape(NWORK * 16, 128)
    pos, val, bins = rank_call(te2d, tpe, cumx)
    indices, bin_ids = permute_call(pos, val)
    return (laux[0, 0], indices, bin_ids, bins, gate.reshape(-1), tpe[0])
